# Initial kernel scaffold; baseline (speedup 1.0000x reference)
import jax
import jax.numpy as jnp
from jax.experimental import pallas as pl


def kernel(vertices):
    raise NotImplementedError("write your pallas kernel here")


# speedup vs baseline: 3.2629x; 3.2629x over previous
import jax
import jax.numpy as jnp
from jax import lax
from jax.experimental import pallas as pl
from jax.experimental.pallas import tpu as pltpu
from jax.experimental.pallas import tpu_sc as plsc

_NB = 16
_N = 4096
_NCH = _N // 16
_CUTS = (1023, 2047, 3071)


def _sc_body(verts, out, cols, ubuf, h1, c1, h2, c2, h3, c3, keybuf, offs, outv):
    cid = lax.axis_index("c")
    sid = lax.axis_index("s")
    wid = sid * 2 + cid

    @pl.when(wid < _NB)
    def _():
        zeros16 = jnp.zeros((16,), jnp.int32)

        cnt0, _ = plsc.scan_count(zeros16)
        base0 = jnp.min(cnt0)
        ones16 = jnp.full((16,), 1, jnp.int32)
        basec = jnp.max(plsc.cumsum(ones16)) - 15

        def exclc(x):
            return plsc.cumsum(x) - x * basec

        pltpu.sync_copy(verts.at[wid], cols)

        q_all = [[None] * 3 for _ in range(3)]
        m3_all = [[None] * 3 for _ in range(3)]

        for a in range(3):
            for ref, n in ((h1, 128), (h2, 384), (h3, 192)):
                def zb(i, _, ref=ref):
                    ref[pl.ds(i * 16, 16)] = zeros16
                    return 0
                lax.fori_loop(0, n, zb, 0)

            def p1(i, _):
                x = cols[a, pl.ds(i * 16, 16)]
                bits = plsc.bitcast(x, jnp.int32)
                skey = bits ^ ((bits >> 31) & jnp.int32(0x7FFFFFFF))
                skey = jnp.where(skey == -1, 0, skey)
                ubuf[a, pl.ds(i * 16, 16)] = skey
                b1 = ((skey >> 21) & 2047) ^ 1024
                cnt, last = plsc.scan_count(b1)
                plsc.addupdate_scatter(h1, [b1], cnt - base0 + 1, mask=last)
                return 0
            lax.fori_loop(0, _NCH, p1, 0)

            def s1(i, carry):
                tot, a0, a1, a2 = carry
                h = h1[pl.ds(i * 16, 16)]
                ex = exclc(h) + tot
                c1[pl.ds(i * 16, 16)] = ex
                a0 = a0 + plsc.all_reduce_population_count(ex <= _CUTS[0])
                a1 = a1 + plsc.all_reduce_population_count(ex <= _CUTS[1])
                a2 = a2 + plsc.all_reduce_population_count(ex <= _CUTS[2])
                return (tot + jnp.sum(h), a0, a1, a2)
            _, a0, a1, a2 = lax.fori_loop(
                0, 128, s1, (jnp.int32(0), zeros16, zeros16, zeros16))
            t1s = [jnp.max(v) - 1 for v in (a0, a1, a2)]
            L1s = [jnp.max(plsc.load_gather(c1, [jnp.full((16,), t, jnp.int32)]))
                   for t in t1s]
            m1s = [jnp.int32(_CUTS[k]) - L1s[k] for k in range(3)]

            reg_b = jnp.where(t1s[1] != t1s[0], 1, 0)
            reg_c = reg_b + jnp.where(t1s[2] != t1s[1], 1, 0)
            regs2 = [jnp.int32(0), reg_b, reg_c]

            def p2(i, _):
                skey = ubuf[a, pl.ds(i * 16, 16)]
                b1 = ((skey >> 21) & 2047) ^ 1024
                b2 = (skey >> 10) & 2047
                m0 = b1 == t1s[0]
                m1 = b1 == t1s[1]
                m2 = b1 == t1s[2]
                ridx = jnp.where(m0, regs2[0], jnp.where(m1, regs2[1], regs2[2]))
                anym = m0 | m1 | m2
                idx = b2 + ridx * 2048
                cnt, last = plsc.scan_count(idx, anym)
                plsc.addupdate_scatter(h2, [idx], cnt - base0 + 1, mask=last & anym)
                return 0
            lax.fori_loop(0, _NCH, p2, 0)

            t2s = []
            m2s = []
            for k in range(3):
                rbase = regs2[k] * 2048
                def s2(i, carry, rbase=rbase, k=k):
                    tot, acc = carry
                    h = h2[pl.ds(rbase + i * 16, 16)]
                    ex = exclc(h) + tot
                    c2[pl.ds(rbase + i * 16, 16)] = ex
                    acc = acc + plsc.all_reduce_population_count(ex <= m1s[k])
                    return (tot + jnp.sum(h), acc)
                _, acc = lax.fori_loop(0, 128, s2, (jnp.int32(0), zeros16))
                t2 = jnp.max(acc) - 1
                L2 = jnp.max(plsc.load_gather(
                    c2, [jnp.full((16,), rbase + t2, jnp.int32)]))
                t2s.append(t2)
                m2s.append(m1s[k] - L2)

            pref22 = [((t1s[k] ^ 1024) << 11) | t2s[k] for k in range(3)]
            reg_b3 = jnp.where(pref22[1] != pref22[0], 1, 0)
            reg_c3 = reg_b3 + jnp.where(pref22[2] != pref22[1], 1, 0)
            regs3 = [jnp.int32(0), reg_b3, reg_c3]

            def p3(i, _):
                skey = ubuf[a, pl.ds(i * 16, 16)]
                hi22 = (skey >> 10) & jnp.int32(0x3FFFFF)
                b3v = skey & 1023
                m0 = hi22 == pref22[0]
                m1 = hi22 == pref22[1]
                m2 = hi22 == pref22[2]
                ridx = jnp.where(m0, regs3[0], jnp.where(m1, regs3[1], regs3[2]))
                anym = m0 | m1 | m2
                idx = b3v + ridx * 1024
                cnt, last = plsc.scan_count(idx, anym)
                plsc.addupdate_scatter(h3, [idx], cnt - base0 + 1, mask=last & anym)
                return 0
            lax.fori_loop(0, _NCH, p3, 0)

            for k in range(3):
                rbase = regs3[k] * 1024
                def s3(i, carry, rbase=rbase, k=k):
                    tot, acc = carry
                    h = h3[pl.ds(rbase + i * 16, 16)]
                    ex = exclc(h) + tot
                    c3[pl.ds(rbase + i * 16, 16)] = ex
                    acc = acc + plsc.all_reduce_population_count(ex <= m2s[k])
                    return (tot + jnp.sum(h), acc)
                _, acc = lax.fori_loop(0, 64, s3, (jnp.int32(0), zeros16))
                t3 = jnp.max(acc) - 1
                L3 = jnp.max(plsc.load_gather(
                    c3, [jnp.full((16,), rbase + t3, jnp.int32)]))
                m3_all[a][k] = m2s[k] - L3
                q_all[a][k] = (pref22[k] << 10) | t3

        def zo(i, _):
            offs[pl.ds(i * 16, 16)] = zeros16
            return 0
        lax.fori_loop(0, 4, zo, 0)

        def pc(i, carry):
            carry = list(carry)
            key = zeros16
            for a in range(3):
                skey = ubuf[a, pl.ds(i * 16, 16)]
                binv = zeros16
                for k in range(3):
                    q = q_all[a][k]
                    m3 = m3_all[a][k]
                    lt = skey < q
                    eqm = skey == q
                    eqi = eqm.astype(jnp.int32)
                    pre = exclc(eqi) + carry[a * 3 + k]
                    lower = lt | (eqm & (pre <= m3))
                    binv = binv + (1 - lower.astype(jnp.int32))
                    carry[a * 3 + k] = carry[a * 3 + k] + jnp.sum(eqi)
                key = key + binv * (1, 4, 16)[a]
            keybuf[pl.ds(i * 16, 16)] = key
            cnt, last = plsc.scan_count(key)
            plsc.addupdate_scatter(offs, [key], cnt - base0 + 1, mask=last)
            return tuple(carry)
        lax.fori_loop(0, _NCH, pc, (jnp.int32(0),) * 9)

        def oc(i, tot):
            h = offs[pl.ds(i * 16, 16)]
            offs[pl.ds(i * 16, 16)] = exclc(h) + tot
            return tot + jnp.sum(h)
        lax.fori_loop(0, 4, oc, jnp.int32(0))

        def pf(i, _):
            key = keybuf[pl.ds(i * 16, 16)]
            cnt, last = plsc.scan_count(key)
            cz = cnt - base0
            basev = plsc.load_gather(offs, [key])
            pos3 = (basev + cz) * 3
            plsc.addupdate_scatter(offs, [key], cz + 1, mask=last)
            plsc.store_scatter(outv, [pos3], cols[0, pl.ds(i * 16, 16)])
            plsc.store_scatter(outv, [pos3 + 1], cols[1, pl.ds(i * 16, 16)])
            plsc.store_scatter(outv, [pos3 + 2], cols[2, pl.ds(i * 16, 16)])
            return 0
        lax.fori_loop(0, _NCH, pf, 0)

        pltpu.sync_copy(outv, out.at[wid])


def kernel(vertices):
    verts_t = vertices.transpose(0, 2, 1)
    f = pl.kernel(
        _sc_body,
        out_type=jax.ShapeDtypeStruct((_NB, _N * 3), jnp.float32),
        compiler_params=pltpu.CompilerParams(needs_layout_passes=False),
        mesh=plsc.VectorSubcoreMesh(
            core_axis_name="c", subcore_axis_name="s",
            num_cores=2, num_subcores=16),
        scratch_types=[
            pltpu.VMEM((3, _N), jnp.float32),
            pltpu.VMEM((3, _N), jnp.int32),
            pltpu.VMEM((2048,), jnp.int32),
            pltpu.VMEM((2048,), jnp.int32),
            pltpu.VMEM((6144,), jnp.int32),
            pltpu.VMEM((6144,), jnp.int32),
            pltpu.VMEM((3072,), jnp.int32),
            pltpu.VMEM((3072,), jnp.int32),
            pltpu.VMEM((_N,), jnp.int32),
            pltpu.VMEM((64,), jnp.int32),
            pltpu.VMEM((_N * 3,), jnp.float32),
        ],
    )
    out = f(verts_t)
    return out.reshape(_NB, 64, 192)
